# baseline (speedup 1.0000x reference)
import functools

import jax
import jax.numpy as jnp
from jax import lax
from jax.experimental import pallas as pl
from jax.experimental.pallas import tpu as pltpu
from jax.experimental.pallas import tpu_sc as plsc

VOCAB = 100000
EMB = 32
CTX = 20
B = 1024
HID = 128
DIN = CTX * EMB

NC, NS = 2, 16
NW = NC * NS
NIDX = B * CTX
IDX_PER_W = NIDX // NW

VT = 1024
NV = (VOCAB + VT - 1) // VT
EDGE = VOCAB - (NV - 1) * VT


def _gather_body(table_hbm, idx_hbm, out_hbm, idx_v, rows_v, sem):
    wid = lax.axis_index("s") * NC + lax.axis_index("c")
    base = wid * IDX_PER_W
    pltpu.sync_copy(idx_hbm.at[pl.ds(base, IDX_PER_W)], idx_v)
    pltpu.async_copy(table_hbm.at[idx_v], rows_v, sem).wait()
    pltpu.sync_copy(rows_v, out_hbm.at[pl.ds(base, IDX_PER_W)])


def _make_gather():
    return functools.partial(
        pl.kernel,
        mesh=plsc.VectorSubcoreMesh(core_axis_name="c", subcore_axis_name="s"),
        out_type=jax.ShapeDtypeStruct((NIDX, EMB), jnp.float32),
        scratch_types=[
            pltpu.VMEM((IDX_PER_W,), jnp.int32),
            pltpu.VMEM((IDX_PER_W, EMB), jnp.float32),
            pltpu.SemaphoreType.DMA,
        ],
        compiler_params=pltpu.CompilerParams(use_tc_tiling_on_sc=False),
    )(_gather_body)


def _mlp_lsm_body(emb_ref, w1_ref, b1_ref, w2_ref, b2_ref, out_ref,
                  h_ref, buf_ref, m_ref, s_ref):
    p = pl.program_id(0)
    j = pl.program_id(1)

    @pl.when((p == 0) & (j == 0))
    def _init():
        h = lax.dot_general(emb_ref[...], w1_ref[...], (((1,), (1,)), ((), ())),
                            preferred_element_type=jnp.float32)
        h_ref[...] = jnp.maximum(h + b1_ref[...], 0.0).astype(jnp.bfloat16)
        m_ref[...] = jnp.full((B, 1), -1e30, jnp.float32)
        s_ref[...] = jnp.zeros((B, 1), jnp.float32)

    @pl.when(j < NV)
    def _matmul():
        logits = lax.dot_general(h_ref[...], w2_ref[...].astype(jnp.bfloat16),
                                 (((1,), (1,)), ((), ())),
                                 preferred_element_type=jnp.float32)
        buf_ref[j % 2] = logits + b2_ref[...]

    def _update_stats(prev):
        tmax = jnp.max(prev, axis=1, keepdims=True)
        mnew = jnp.maximum(m_ref[...], tmax)
        s_ref[...] = (s_ref[...] * jnp.exp(m_ref[...] - mnew)
                      + jnp.sum(jnp.exp(prev - mnew), axis=1, keepdims=True))
        m_ref[...] = mnew

    @pl.when((p == 0) & (j > 0) & (j < NV))
    def _stats():
        _update_stats(buf_ref[(j - 1) % 2])

    @pl.when((p == 0) & (j == NV))
    def _stats_edge():
        mask = lax.broadcasted_iota(jnp.int32, (1, VT), 1) < EDGE
        _update_stats(jnp.where(mask, buf_ref[(j - 1) % 2], -1e30))

    @pl.when((p == 1) & (j == 0))
    def _finalize_lse():
        m_ref[...] = m_ref[...] + jnp.log(s_ref[...])

    @pl.when((p == 1) & (j > 0))
    def _emit():
        out_ref[...] = buf_ref[(j - 1) % 2] - m_ref[...]


def kernel(context_words, emb_table, W1, b1, W2, b2):
    idx = context_words.reshape(NIDX)
    rows = _make_gather()(emb_table, idx)
    emb = rows.reshape(B, DIN)

    out = pl.pallas_call(
        _mlp_lsm_body,
        grid=(2, NV + 1),
        in_specs=[
            pl.BlockSpec((B, DIN), lambda p, j: (0, 0)),
            pl.BlockSpec((HID, DIN), lambda p, j: (0, 0)),
            pl.BlockSpec((1, HID), lambda p, j: (0, 0)),
            pl.BlockSpec((VT, HID), lambda p, j: (jnp.minimum(j, NV - 1), 0)),
            pl.BlockSpec((1, VT), lambda p, j: (0, jnp.minimum(j, NV - 1))),
        ],
        out_specs=pl.BlockSpec(
            (B, VT),
            lambda p, j: (0, jnp.where(p == 1, jnp.maximum(j - 1, 0), 0))),
        out_shape=jax.ShapeDtypeStruct((B, VOCAB), jnp.float32),
        scratch_shapes=[
            pltpu.VMEM((B, HID), jnp.bfloat16),
            pltpu.VMEM((2, B, VT), jnp.float32),
            pltpu.VMEM((B, 1), jnp.float32),
            pltpu.VMEM((B, 1), jnp.float32),
        ],
        compiler_params=pltpu.CompilerParams(
            dimension_semantics=("arbitrary", "arbitrary")),
    )(emb, W1, b1.reshape(1, HID), W2, b2.reshape(1, VOCAB))
    return out


# speedup vs baseline: 1.0025x; 1.0025x over previous
import functools

import jax
import jax.numpy as jnp
from jax import lax
from jax.experimental import pallas as pl
from jax.experimental.pallas import tpu as pltpu
from jax.experimental.pallas import tpu_sc as plsc

VOCAB = 100000
EMB = 32
CTX = 20
B = 1024
HID = 128
GRP = 128 // EMB
DINW = CTX * 128

NC, NS = 2, 16
NW = NC * NS
NIDX = B * CTX
IDX_PER_W = NIDX // NW

VT = 1024
NV = (VOCAB + VT - 1) // VT
EDGE = VOCAB - (NV - 1) * VT


def _gather_body(table_hbm, idx_hbm, out_hbm, idx_v, rows_v, sem):
    wid = lax.axis_index("s") * NC + lax.axis_index("c")
    base = wid * IDX_PER_W
    pltpu.sync_copy(idx_hbm.at[pl.ds(base, IDX_PER_W)], idx_v)
    pltpu.async_copy(table_hbm.at[idx_v], rows_v, sem).wait()
    pltpu.sync_copy(rows_v, out_hbm.at[pl.ds(base, IDX_PER_W)])


def _make_gather():
    return functools.partial(
        pl.kernel,
        mesh=plsc.VectorSubcoreMesh(core_axis_name="c", subcore_axis_name="s"),
        out_type=jax.ShapeDtypeStruct((NIDX, 128), jnp.float32),
        scratch_types=[
            pltpu.VMEM((IDX_PER_W,), jnp.int32),
            pltpu.VMEM((IDX_PER_W, 128), jnp.float32),
            pltpu.SemaphoreType.DMA,
        ],
    )(_gather_body)


def _mlp_lsm_body(embw_ref, mask_ref, w1w_ref, b1_ref, w2_ref, b2_ref, out_ref,
                  h_ref, bufa_ref, bufb_ref, m_ref, s_ref):
    p = pl.program_id(0)
    j = pl.program_id(1)

    @pl.when((p == 0) & (j == 0))
    def _init():
        e = embw_ref[...] * mask_ref[...]
        h = lax.dot_general(e, w1w_ref[...], (((1,), (1,)), ((), ())),
                            preferred_element_type=jnp.float32)
        h_ref[...] = jnp.maximum(h + b1_ref[...], 0.0).astype(jnp.bfloat16)
        m_ref[...] = jnp.full((B, 1), -1e30, jnp.float32)
        s_ref[...] = jnp.zeros((B, 1), jnp.float32)

    def _matmul(dst_ref):
        logits = lax.dot_general(h_ref[...], w2_ref[...].astype(jnp.bfloat16),
                                 (((1,), (1,)), ((), ())),
                                 preferred_element_type=jnp.float32)
        dst_ref[...] = logits + b2_ref[...]

    def _stats(src_ref):
        col_ok = lax.broadcasted_iota(jnp.int32, (1, VT), 1) < EDGE
        valid = (j > 0) & ((j < NV) | col_ok)
        prev = jnp.where(valid, src_ref[...], -1e30)
        tmax = jnp.max(prev, axis=1, keepdims=True)
        mnew = jnp.maximum(m_ref[...], tmax)
        s_ref[...] = (s_ref[...] * jnp.exp(m_ref[...] - mnew)
                      + jnp.sum(jnp.exp(prev - mnew), axis=1, keepdims=True))
        m_ref[...] = mnew

    def _emit(src_ref):
        out_ref[...] = src_ref[...] - (m_ref[...] + jnp.log(s_ref[...]))

    even = j % 2 == 0

    @pl.when((p == 0) & even)
    def _p0_even():
        _matmul(bufa_ref)
        _stats(bufb_ref)

    @pl.when((p == 0) & ~even)
    def _p0_odd():
        _matmul(bufb_ref)
        _stats(bufa_ref)

    @pl.when((p == 1) & even)
    def _p1_even():
        _matmul(bufa_ref)
        _emit(bufb_ref)

    @pl.when((p == 1) & ~even)
    def _p1_odd():
        _matmul(bufb_ref)
        _emit(bufa_ref)


def kernel(context_words, emb_table, W1, b1, W2, b2):
    idx = context_words.reshape(NIDX)
    table_g = emb_table.reshape(VOCAB // GRP, 128)
    rows = _make_gather()(table_g, idx // GRP)
    embw = rows.reshape(B, DINW)

    ksel = (context_words % GRP).astype(jnp.int32)
    onehot = (ksel[:, :, None] == jnp.arange(GRP, dtype=jnp.int32))
    mask = jnp.broadcast_to(onehot[:, :, :, None].astype(jnp.float32),
                            (B, CTX, GRP, EMB)).reshape(B, DINW)
    w1w = jnp.broadcast_to(
        W1.reshape(HID, CTX, 1, EMB), (HID, CTX, GRP, EMB)).reshape(HID, DINW)

    out = pl.pallas_call(
        _mlp_lsm_body,
        grid=(2, NV + 1),
        in_specs=[
            pl.BlockSpec((B, DINW), lambda p, j: (0, 0)),
            pl.BlockSpec((B, DINW), lambda p, j: (0, 0)),
            pl.BlockSpec((HID, DINW), lambda p, j: (0, 0)),
            pl.BlockSpec((1, HID), lambda p, j: (0, 0)),
            pl.BlockSpec((VT, HID), lambda p, j: (jnp.minimum(j, NV - 1), 0)),
            pl.BlockSpec((1, VT), lambda p, j: (0, jnp.minimum(j, NV - 1))),
        ],
        out_specs=pl.BlockSpec(
            (B, VT),
            lambda p, j: (0, jnp.where(p == 1, jnp.maximum(j - 1, 0), 0))),
        out_shape=jax.ShapeDtypeStruct((B, VOCAB), jnp.float32),
        scratch_shapes=[
            pltpu.VMEM((B, HID), jnp.bfloat16),
            pltpu.VMEM((B, VT), jnp.float32),
            pltpu.VMEM((B, VT), jnp.float32),
            pltpu.VMEM((B, 1), jnp.float32),
            pltpu.VMEM((B, 1), jnp.float32),
        ],
        compiler_params=pltpu.CompilerParams(
            dimension_semantics=("arbitrary", "arbitrary")),
    )(embw, mask, w1w, b1.reshape(1, HID), W2, b2.reshape(1, VOCAB))
    return out


# speedup vs baseline: 1.5258x; 1.5221x over previous
import functools

import jax
import jax.numpy as jnp
from jax import lax
from jax.experimental import pallas as pl
from jax.experimental.pallas import tpu as pltpu
from jax.experimental.pallas import tpu_sc as plsc

VOCAB = 100000
EMB = 32
CTX = 20
B = 1024
HID = 128
GRP = 128 // EMB
DINW = CTX * 128

NC, NS = 2, 16
NW = NC * NS
NIDX = B * CTX
IDX_PER_W = NIDX // NW

VT = 1024
NV = (VOCAB + VT - 1) // VT
EDGE = VOCAB - (NV - 1) * VT


def _gather_body(table_hbm, idx_hbm, out_hbm, idx_v, rows_v, sem):
    wid = lax.axis_index("s") * NC + lax.axis_index("c")
    base = wid * IDX_PER_W
    pltpu.sync_copy(idx_hbm.at[pl.ds(base, IDX_PER_W)], idx_v)
    pltpu.async_copy(table_hbm.at[idx_v], rows_v, sem).wait()
    pltpu.sync_copy(rows_v, out_hbm.at[pl.ds(base, IDX_PER_W)])


def _make_gather():
    return functools.partial(
        pl.kernel,
        mesh=plsc.VectorSubcoreMesh(core_axis_name="c", subcore_axis_name="s"),
        out_type=jax.ShapeDtypeStruct((NIDX, 128), jnp.float32),
        scratch_types=[
            pltpu.VMEM((IDX_PER_W,), jnp.int32),
            pltpu.VMEM((IDX_PER_W, 128), jnp.float32),
            pltpu.SemaphoreType.DMA,
        ],
    )(_gather_body)


def _mlp_lsm_body(embw_ref, mask_ref, w1w_ref, b1_ref, w2_ref, b2_ref, out_ref,
                  h_ref, bufa_ref, bufb_ref, m_ref, s_ref):
    p = pl.program_id(0)
    j = pl.program_id(1)

    @pl.when((p == 0) & (j == 0))
    def _init():
        e = embw_ref[...] * mask_ref[...]
        h = lax.dot_general(e, w1w_ref[...], (((1,), (1,)), ((), ())),
                            preferred_element_type=jnp.float32)
        h_ref[...] = jnp.maximum(h + b1_ref[...], 0.0).astype(jnp.bfloat16)
        m_ref[...] = jnp.full((1, B), -1e30, jnp.float32)
        s_ref[...] = jnp.zeros((1, B), jnp.float32)

    def _matmul(dst_ref):
        logits = lax.dot_general(w2_ref[...].astype(jnp.bfloat16), h_ref[...],
                                 (((1,), (1,)), ((), ())),
                                 preferred_element_type=jnp.float32)
        dst_ref[...] = logits + b2_ref[...]

    def _stats(src_ref):
        thresh = jnp.where(j == 0, 0, jnp.where(j <= NV - 1, VT, EDGE))
        row_ok = lax.broadcasted_iota(jnp.int32, (VT, B), 0) < thresh
        prev = jnp.where(row_ok, src_ref[...], -1e30)
        tmax = jnp.max(prev, axis=0, keepdims=True)
        mnew = jnp.maximum(m_ref[...], tmax)
        s_ref[...] = (s_ref[...] * jnp.exp(m_ref[...] - mnew)
                      + jnp.sum(jnp.exp(prev - mnew), axis=0, keepdims=True))
        m_ref[...] = mnew

    def _emit(src_ref):
        out_ref[...] = src_ref[...] - (m_ref[...] + jnp.log(s_ref[...]))

    even = j % 2 == 0

    @pl.when((p == 0) & even)
    def _p0_even():
        _matmul(bufa_ref)
        _stats(bufb_ref)

    @pl.when((p == 0) & ~even)
    def _p0_odd():
        _matmul(bufb_ref)
        _stats(bufa_ref)

    @pl.when((p == 1) & even)
    def _p1_even():
        _matmul(bufa_ref)
        _emit(bufb_ref)

    @pl.when((p == 1) & ~even)
    def _p1_odd():
        _matmul(bufb_ref)
        _emit(bufa_ref)


def kernel(context_words, emb_table, W1, b1, W2, b2):
    idx = context_words.reshape(NIDX)
    table_g = emb_table.reshape(VOCAB // GRP, 128)
    rows = _make_gather()(table_g, idx // GRP)
    embw = rows.reshape(B, DINW)

    ksel = (context_words % GRP).astype(jnp.int32)
    onehot = (ksel[:, :, None] == jnp.arange(GRP, dtype=jnp.int32))
    mask = jnp.broadcast_to(onehot[:, :, :, None].astype(jnp.float32),
                            (B, CTX, GRP, EMB)).reshape(B, DINW)
    w1w = jnp.broadcast_to(
        W1.reshape(HID, CTX, 1, EMB), (HID, CTX, GRP, EMB)).reshape(HID, DINW)

    out_t = pl.pallas_call(
        _mlp_lsm_body,
        grid=(2, NV + 1),
        in_specs=[
            pl.BlockSpec((B, DINW), lambda p, j: (0, 0)),
            pl.BlockSpec((B, DINW), lambda p, j: (0, 0)),
            pl.BlockSpec((HID, DINW), lambda p, j: (0, 0)),
            pl.BlockSpec((1, HID), lambda p, j: (0, 0)),
            pl.BlockSpec((VT, HID), lambda p, j: (jnp.minimum(j, NV - 1), 0)),
            pl.BlockSpec((VT, 1), lambda p, j: (jnp.minimum(j, NV - 1), 0)),
        ],
        out_specs=pl.BlockSpec(
            (VT, B),
            lambda p, j: (jnp.where(p == 1, jnp.maximum(j - 1, 0), 0), 0)),
        out_shape=jax.ShapeDtypeStruct((VOCAB, B), jnp.float32),
        scratch_shapes=[
            pltpu.VMEM((B, HID), jnp.bfloat16),
            pltpu.VMEM((VT, B), jnp.float32),
            pltpu.VMEM((VT, B), jnp.float32),
            pltpu.VMEM((1, B), jnp.float32),
            pltpu.VMEM((1, B), jnp.float32),
        ],
        compiler_params=pltpu.CompilerParams(
            dimension_semantics=("arbitrary", "arbitrary")),
    )(embw, mask, w1w, b1.reshape(1, HID), W2, b2.reshape(VOCAB, 1))
    return out_t.T


# speedup vs baseline: 1.6968x; 1.1120x over previous
import functools

import jax
import jax.numpy as jnp
from jax import lax
from jax.experimental import pallas as pl
from jax.experimental.pallas import tpu as pltpu
from jax.experimental.pallas import tpu_sc as plsc

VOCAB = 100000
EMB = 32
CTX = 20
B = 1024
HID = 128
DIN = CTX * EMB

NC, NS = 2, 16
NW = NC * NS
NELEM = B * DIN
ELEM_PER_W = NELEM // NW

VT = 1024
NV = (VOCAB + VT - 1) // VT
EDGE = VOCAB - (NV - 1) * VT


def _gather_body(table_hbm, idx_hbm, out_hbm, idx_v, vals_v, sem):
    wid = lax.axis_index("s") * NC + lax.axis_index("c")
    base = wid * ELEM_PER_W
    pltpu.sync_copy(idx_hbm.at[pl.ds(base, ELEM_PER_W)], idx_v)
    pltpu.async_copy(table_hbm.at[idx_v], vals_v, sem).wait()
    pltpu.sync_copy(vals_v, out_hbm.at[pl.ds(base, ELEM_PER_W)])


def _make_gather():
    return functools.partial(
        pl.kernel,
        mesh=plsc.VectorSubcoreMesh(core_axis_name="c", subcore_axis_name="s"),
        out_type=jax.ShapeDtypeStruct((NELEM,), jnp.float32),
        scratch_types=[
            pltpu.VMEM((ELEM_PER_W,), jnp.int32),
            pltpu.VMEM((ELEM_PER_W,), jnp.float32),
            pltpu.SemaphoreType.DMA,
        ],
    )(_gather_body)


def _mlp_lsm_body(emb_ref, w1_ref, b1_ref, w2_ref, b2_ref, out_ref,
                  h_ref, bufa_ref, bufb_ref, m_ref, s_ref):
    p = pl.program_id(0)
    j = pl.program_id(1)

    @pl.when((p == 0) & (j == 0))
    def _init():
        h = lax.dot_general(emb_ref[...], w1_ref[...], (((1,), (1,)), ((), ())),
                            preferred_element_type=jnp.float32)
        h_ref[...] = jnp.maximum(h + b1_ref[...], 0.0).astype(jnp.bfloat16)
        m_ref[...] = jnp.full((1, B), -1e30, jnp.float32)
        s_ref[...] = jnp.zeros((1, B), jnp.float32)

    def _matmul(dst_ref):
        logits = lax.dot_general(w2_ref[...].astype(jnp.bfloat16), h_ref[...],
                                 (((1,), (1,)), ((), ())),
                                 preferred_element_type=jnp.float32)
        dst_ref[...] = logits + b2_ref[...].reshape(VT, 1)

    def _stats(src_ref):
        thresh = jnp.where(j == 0, 0, jnp.where(j <= NV - 1, VT, EDGE))
        row_ok = lax.broadcasted_iota(jnp.int32, (VT, B), 0) < thresh
        prev = jnp.where(row_ok, src_ref[...], -1e30)
        tmax = jnp.max(prev, axis=0, keepdims=True)
        mnew = jnp.maximum(m_ref[...], tmax)
        s_ref[...] = (s_ref[...] * jnp.exp(m_ref[...] - mnew)
                      + jnp.sum(jnp.exp(prev - mnew), axis=0, keepdims=True))
        m_ref[...] = mnew

    def _emit(src_ref):
        out_ref[...] = src_ref[...] - (m_ref[...] + jnp.log(s_ref[...]))

    even = j % 2 == 0

    @pl.when((p == 0) & even)
    def _p0_even():
        _matmul(bufa_ref)
        _stats(bufb_ref)

    @pl.when((p == 0) & ~even)
    def _p0_odd():
        _matmul(bufb_ref)
        _stats(bufa_ref)

    @pl.when((p == 1) & even)
    def _p1_even():
        _matmul(bufa_ref)
        _emit(bufb_ref)

    @pl.when((p == 1) & ~even)
    def _p1_odd():
        _matmul(bufb_ref)
        _emit(bufa_ref)


def kernel(context_words, emb_table, W1, b1, W2, b2):
    idx = context_words.reshape(B * CTX)
    idx32 = (idx[:, None] * EMB
             + jnp.arange(EMB, dtype=jnp.int32)).reshape(NELEM)
    table_flat = emb_table.reshape(VOCAB * EMB)
    flat = _make_gather()(table_flat, idx32)
    emb = flat.reshape(B, DIN)

    out_t = pl.pallas_call(
        _mlp_lsm_body,
        grid=(2, NV + 1),
        in_specs=[
            pl.BlockSpec((B, DIN), lambda p, j: (0, 0)),
            pl.BlockSpec((HID, DIN), lambda p, j: (0, 0)),
            pl.BlockSpec((1, HID), lambda p, j: (0, 0)),
            pl.BlockSpec((VT, HID), lambda p, j: (jnp.minimum(j, NV - 1), 0)),
            pl.BlockSpec((1, VT), lambda p, j: (0, jnp.minimum(j, NV - 1))),
        ],
        out_specs=pl.BlockSpec(
            (VT, B),
            lambda p, j: (jnp.where(p == 1, jnp.maximum(j - 1, 0), 0), 0)),
        out_shape=jax.ShapeDtypeStruct((VOCAB, B), jnp.float32),
        scratch_shapes=[
            pltpu.VMEM((B, HID), jnp.bfloat16),
            pltpu.VMEM((VT, B), jnp.float32),
            pltpu.VMEM((VT, B), jnp.float32),
            pltpu.VMEM((1, B), jnp.float32),
            pltpu.VMEM((1, B), jnp.float32),
        ],
        compiler_params=pltpu.CompilerParams(
            dimension_semantics=("arbitrary", "arbitrary")),
    )(emb, W1, b1.reshape(1, HID), W2, b2.reshape(1, VOCAB))
    return out_t.T


# speedup vs baseline: 1.7500x; 1.0314x over previous
import functools

import jax
import jax.numpy as jnp
from jax import lax
from jax.experimental import pallas as pl
from jax.experimental.pallas import tpu as pltpu
from jax.experimental.pallas import tpu_sc as plsc

VOCAB = 100000
EMB = 32
CTX = 20
B = 1024
HID = 128
DIN = CTX * EMB

NC, NS = 2, 16
NW = NC * NS
NELEM = B * DIN
ELEM_PER_W = NELEM // NW

VT = 1024
NV = (VOCAB + VT - 1) // VT
EDGE = VOCAB - (NV - 1) * VT


def _gather_body(table_hbm, idx_hbm, out_hbm, idx_v, vals_v, sem):
    wid = lax.axis_index("s") * NC + lax.axis_index("c")
    base = wid * ELEM_PER_W
    pltpu.sync_copy(idx_hbm.at[pl.ds(base, ELEM_PER_W)], idx_v)
    pltpu.async_copy(table_hbm.at[idx_v], vals_v, sem).wait()
    pltpu.sync_copy(vals_v, out_hbm.at[pl.ds(base, ELEM_PER_W)])


def _make_gather():
    return functools.partial(
        pl.kernel,
        mesh=plsc.VectorSubcoreMesh(core_axis_name="c", subcore_axis_name="s"),
        out_type=jax.ShapeDtypeStruct((NELEM,), jnp.float32),
        scratch_types=[
            pltpu.VMEM((ELEM_PER_W,), jnp.int32),
            pltpu.VMEM((ELEM_PER_W,), jnp.float32),
            pltpu.SemaphoreType.DMA,
        ],
    )(_gather_body)


def _mlp_lsm_body(emb_ref, w1_ref, b1_ref, w2_ref, b2_ref, out_ref,
                  h_ref, bufa_ref, bufb_ref, tma_ref, tmb_ref, m_ref, s_ref):
    p = pl.program_id(0)
    j = pl.program_id(1)

    @pl.when((p == 0) & (j == 0))
    def _init():
        h = lax.dot_general(emb_ref[...], w1_ref[...], (((1,), (1,)), ((), ())),
                            preferred_element_type=jnp.float32)
        h_ref[...] = jnp.maximum(h + b1_ref[...], 0.0).astype(jnp.bfloat16)
        m_ref[...] = jnp.full((1, B), -1e30, jnp.float32)
        s_ref[...] = jnp.zeros((1, B), jnp.float32)
        bufb_ref[...] = jnp.full((VT, B), -jnp.inf, jnp.float32)
        tmb_ref[...] = jnp.full((1, B), -1e30, jnp.float32)

    def _logits():
        mm = lax.dot_general(w2_ref[...].astype(jnp.bfloat16), h_ref[...],
                             (((1,), (1,)), ((), ())),
                             preferred_element_type=jnp.float32)
        return mm + b2_ref[...].reshape(VT, 1)

    def _produce(dst_ref, tm_ref, masked):
        logits = _logits()
        if masked:
            row_ok = lax.broadcasted_iota(jnp.int32, (VT, B), 0) < EDGE
            logits = jnp.where(row_ok, logits, -1e30)
        dst_ref[...] = logits
        tm_ref[...] = jnp.max(logits, axis=0, keepdims=True)

    def _stats(src_ref, tm_ref):
        mnew = jnp.maximum(m_ref[...], tm_ref[...])
        s_ref[...] = (s_ref[...] * jnp.exp(m_ref[...] - mnew)
                      + jnp.sum(jnp.exp(src_ref[...] - mnew), axis=0,
                                keepdims=True))
        m_ref[...] = mnew

    even = j % 2 == 0

    @pl.when((p == 0) & even)
    def _p0_even():
        _produce(bufa_ref, tma_ref, masked=False)
        _stats(bufb_ref, tmb_ref)

    @pl.when((p == 0) & ~even & (j != NV - 1))
    def _p0_odd():
        _produce(bufb_ref, tmb_ref, masked=False)
        _stats(bufa_ref, tma_ref)

    @pl.when((p == 0) & (j == NV - 1))
    def _p0_edge():
        _produce(bufb_ref, tmb_ref, masked=True)
        _stats(bufa_ref, tma_ref)

    @pl.when((p == 1) & (j == 0))
    def _finalize():
        m_ref[...] = m_ref[...] + jnp.log(s_ref[...])

    @pl.when((p == 1) & (j < NV))
    def _emit():
        out_ref[...] = _logits() - m_ref[...]


def kernel(context_words, emb_table, W1, b1, W2, b2):
    idx = context_words.reshape(B * CTX)
    idx32 = (idx[:, None] * EMB
             + jnp.arange(EMB, dtype=jnp.int32)).reshape(NELEM)
    table_flat = emb_table.reshape(VOCAB * EMB)
    flat = _make_gather()(table_flat, idx32)
    emb = flat.reshape(B, DIN)

    out_t = pl.pallas_call(
        _mlp_lsm_body,
        grid=(2, NV + 1),
        in_specs=[
            pl.BlockSpec((B, DIN), lambda p, j: (0, 0)),
            pl.BlockSpec((HID, DIN), lambda p, j: (0, 0)),
            pl.BlockSpec((1, HID), lambda p, j: (0, 0)),
            pl.BlockSpec((VT, HID), lambda p, j: (jnp.minimum(j, NV - 1), 0)),
            pl.BlockSpec((1, VT), lambda p, j: (0, jnp.minimum(j, NV - 1))),
        ],
        out_specs=pl.BlockSpec(
            (VT, B),
            lambda p, j: (jnp.where(p == 1, jnp.minimum(j, NV - 1), 0), 0)),
        out_shape=jax.ShapeDtypeStruct((VOCAB, B), jnp.float32),
        scratch_shapes=[
            pltpu.VMEM((B, HID), jnp.bfloat16),
            pltpu.VMEM((VT, B), jnp.float32),
            pltpu.VMEM((VT, B), jnp.float32),
            pltpu.VMEM((1, B), jnp.float32),
            pltpu.VMEM((1, B), jnp.float32),
            pltpu.VMEM((1, B), jnp.float32),
            pltpu.VMEM((1, B), jnp.float32),
        ],
        compiler_params=pltpu.CompilerParams(
            dimension_semantics=("arbitrary", "arbitrary")),
    )(emb, W1, b1.reshape(1, HID), W2, b2.reshape(1, VOCAB))
    return out_t.T


# speedup vs baseline: 1.8985x; 1.0848x over previous
import functools

import jax
import jax.numpy as jnp
from jax import lax
from jax.experimental import pallas as pl
from jax.experimental.pallas import tpu as pltpu
from jax.experimental.pallas import tpu_sc as plsc

VOCAB = 100000
EMB = 32
CTX = 20
B = 1024
HID = 128
DIN = CTX * EMB

NC, NS = 2, 16
NW = NC * NS
NELEM = B * DIN
ELEM_PER_W = NELEM // NW

VT = 2048
NV = (VOCAB + VT - 1) // VT
EDGE = VOCAB - (NV - 1) * VT
EDGE_ODD = (NV - 1) % 2 == 1


def _gather_body(table_hbm, idx_hbm, out_hbm, idx_v, vals_v, sem):
    wid = lax.axis_index("s") * NC + lax.axis_index("c")
    base = wid * ELEM_PER_W
    pltpu.sync_copy(idx_hbm.at[pl.ds(base, ELEM_PER_W)], idx_v)
    pltpu.async_copy(table_hbm.at[idx_v], vals_v, sem).wait()
    pltpu.sync_copy(vals_v, out_hbm.at[pl.ds(base, ELEM_PER_W)])


def _make_gather():
    return functools.partial(
        pl.kernel,
        mesh=plsc.VectorSubcoreMesh(core_axis_name="c", subcore_axis_name="s"),
        out_type=jax.ShapeDtypeStruct((NELEM,), jnp.float32),
        scratch_types=[
            pltpu.VMEM((ELEM_PER_W,), jnp.int32),
            pltpu.VMEM((ELEM_PER_W,), jnp.float32),
            pltpu.SemaphoreType.DMA,
        ],
    )(_gather_body)


def _mlp_lsm_body(emb_ref, w1_ref, b1_ref, w2_ref, b2_ref, out_ref,
                  h_ref, bufa_ref, bufb_ref, tma_ref, tmb_ref, m_ref, s_ref):
    p = pl.program_id(0)
    j = pl.program_id(1)

    @pl.when((p == 0) & (j == 0))
    def _init():
        h = lax.dot_general(emb_ref[...], w1_ref[...], (((1,), (1,)), ((), ())),
                            preferred_element_type=jnp.float32)
        h_ref[...] = jnp.maximum(h + b1_ref[...], 0.0).astype(jnp.bfloat16)
        m_ref[...] = jnp.full((1, B), -1e30, jnp.float32)
        s_ref[...] = jnp.zeros((1, B), jnp.float32)
        bufb_ref[...] = jnp.full((VT, B), -jnp.inf, jnp.float32)
        tmb_ref[...] = jnp.full((1, B), -1e30, jnp.float32)

    def _logits():
        mm = lax.dot_general(w2_ref[...].astype(jnp.bfloat16), h_ref[...],
                             (((1,), (1,)), ((), ())),
                             preferred_element_type=jnp.float32)
        return mm + b2_ref[...].reshape(VT, 1)

    def _produce(dst_ref, tm_ref, masked):
        logits = _logits()
        if masked:
            row_ok = lax.broadcasted_iota(jnp.int32, (VT, B), 0) < EDGE
            logits = jnp.where(row_ok, logits, -1e30)
        dst_ref[...] = logits
        tm_ref[...] = jnp.max(logits, axis=0, keepdims=True)

    def _stats(src_ref, tm_ref):
        mnew = jnp.maximum(m_ref[...], tm_ref[...])
        s_ref[...] = (s_ref[...] * jnp.exp(m_ref[...] - mnew)
                      + jnp.sum(jnp.exp(src_ref[...] - mnew), axis=0,
                                keepdims=True))
        m_ref[...] = mnew

    even = j % 2 == 0
    pred_even = (p == 0) & even
    pred_odd = (p == 0) & ~even
    if EDGE_ODD:
        pred_odd = pred_odd & (j != NV - 1)
    else:
        pred_even = pred_even & (j != NV - 1)

    @pl.when(pred_even)
    def _p0_even():
        _produce(bufa_ref, tma_ref, masked=False)
        _stats(bufb_ref, tmb_ref)

    @pl.when(pred_odd)
    def _p0_odd():
        _produce(bufb_ref, tmb_ref, masked=False)
        _stats(bufa_ref, tma_ref)

    @pl.when((p == 0) & (j == NV - 1))
    def _p0_edge():
        if EDGE_ODD:
            _produce(bufb_ref, tmb_ref, masked=True)
            _stats(bufa_ref, tma_ref)
        else:
            _produce(bufa_ref, tma_ref, masked=True)
            _stats(bufb_ref, tmb_ref)

    @pl.when((p == 1) & (j == 0))
    def _finalize():
        m_ref[...] = m_ref[...] + jnp.log(s_ref[...])

    @pl.when((p == 1) & (j < NV))
    def _emit():
        out_ref[...] = _logits() - m_ref[...]


def kernel(context_words, emb_table, W1, b1, W2, b2):
    idx = context_words.reshape(B * CTX)
    idx32 = (idx[:, None] * EMB
             + jnp.arange(EMB, dtype=jnp.int32)).reshape(NELEM)
    table_flat = emb_table.reshape(VOCAB * EMB)
    flat = _make_gather()(table_flat, idx32)
    emb = flat.reshape(B, DIN)

    out_t = pl.pallas_call(
        _mlp_lsm_body,
        grid=(2, NV + 1),
        in_specs=[
            pl.BlockSpec((B, DIN), lambda p, j: (0, 0)),
            pl.BlockSpec((HID, DIN), lambda p, j: (0, 0)),
            pl.BlockSpec((1, HID), lambda p, j: (0, 0)),
            pl.BlockSpec((VT, HID), lambda p, j: (jnp.minimum(j, NV - 1), 0)),
            pl.BlockSpec((1, VT), lambda p, j: (0, jnp.minimum(j, NV - 1))),
        ],
        out_specs=pl.BlockSpec(
            (VT, B),
            lambda p, j: (jnp.where(p == 1, jnp.minimum(j, NV - 1), 0), 0)),
        out_shape=jax.ShapeDtypeStruct((VOCAB, B), jnp.float32),
        scratch_shapes=[
            pltpu.VMEM((B, HID), jnp.bfloat16),
            pltpu.VMEM((VT, B), jnp.float32),
            pltpu.VMEM((VT, B), jnp.float32),
            pltpu.VMEM((1, B), jnp.float32),
            pltpu.VMEM((1, B), jnp.float32),
            pltpu.VMEM((1, B), jnp.float32),
            pltpu.VMEM((1, B), jnp.float32),
        ],
        compiler_params=pltpu.CompilerParams(
            dimension_semantics=("arbitrary", "arbitrary")),
    )(emb, W1, b1.reshape(1, HID), W2, b2.reshape(1, VOCAB))
    return out_t.T


# speedup vs baseline: 1.9223x; 1.0125x over previous
import functools

import jax
import jax.numpy as jnp
from jax import lax
from jax.experimental import pallas as pl
from jax.experimental.pallas import tpu as pltpu
from jax.experimental.pallas import tpu_sc as plsc

VOCAB = 100000
EMB = 32
CTX = 20
B = 1024
HID = 128
DIN = CTX * EMB

NC, NS = 2, 16
NW = NC * NS
NELEM = B * DIN
ELEM_PER_W = NELEM // NW

VT = 2560
NV = (VOCAB + VT - 1) // VT
EDGE = VOCAB - (NV - 1) * VT
EDGE_ODD = (NV - 1) % 2 == 1


def _gather_body(table_hbm, idx_hbm, out_hbm, idx_v, vals_v, sem):
    wid = lax.axis_index("s") * NC + lax.axis_index("c")
    base = wid * ELEM_PER_W
    pltpu.sync_copy(idx_hbm.at[pl.ds(base, ELEM_PER_W)], idx_v)
    pltpu.async_copy(table_hbm.at[idx_v], vals_v, sem).wait()
    pltpu.sync_copy(vals_v, out_hbm.at[pl.ds(base, ELEM_PER_W)])


def _make_gather():
    return functools.partial(
        pl.kernel,
        mesh=plsc.VectorSubcoreMesh(core_axis_name="c", subcore_axis_name="s"),
        out_type=jax.ShapeDtypeStruct((NELEM,), jnp.float32),
        scratch_types=[
            pltpu.VMEM((ELEM_PER_W,), jnp.int32),
            pltpu.VMEM((ELEM_PER_W,), jnp.float32),
            pltpu.SemaphoreType.DMA,
        ],
    )(_gather_body)


def _mlp_lsm_body(emb_ref, w1_ref, b1_ref, w2_ref, b2_ref, out_ref,
                  h_ref, bufa_ref, bufb_ref, tma_ref, tmb_ref, m_ref, s_ref):
    p = pl.program_id(0)
    j = pl.program_id(1)

    @pl.when((p == 0) & (j == 0))
    def _init():
        h = lax.dot_general(emb_ref[...], w1_ref[...], (((1,), (1,)), ((), ())),
                            preferred_element_type=jnp.float32)
        h_ref[...] = jnp.maximum(h + b1_ref[...], 0.0).astype(jnp.bfloat16)
        m_ref[...] = jnp.full((1, B), -1e30, jnp.float32)
        s_ref[...] = jnp.zeros((1, B), jnp.float32)
        bufb_ref[...] = jnp.full((VT, B), -jnp.inf, jnp.float32)
        tmb_ref[...] = jnp.full((1, B), -1e30, jnp.float32)

    def _logits():
        mm = lax.dot_general(w2_ref[...].astype(jnp.bfloat16), h_ref[...],
                             (((1,), (1,)), ((), ())),
                             preferred_element_type=jnp.float32)
        return mm + b2_ref[...].reshape(VT, 1)

    def _produce(dst_ref, tm_ref, masked):
        logits = _logits()
        if masked:
            row_ok = lax.broadcasted_iota(jnp.int32, (VT, B), 0) < EDGE
            logits = jnp.where(row_ok, logits, -1e30)
        dst_ref[...] = logits
        tm_ref[...] = jnp.max(logits, axis=0, keepdims=True)

    def _stats(src_ref, tm_ref):
        mnew = jnp.maximum(m_ref[...], tm_ref[...])
        s_ref[...] = (s_ref[...] * jnp.exp(m_ref[...] - mnew)
                      + jnp.sum(jnp.exp(src_ref[...] - mnew), axis=0,
                                keepdims=True))
        m_ref[...] = mnew

    even = j % 2 == 0
    pred_even = (p == 0) & even
    pred_odd = (p == 0) & ~even
    if EDGE_ODD:
        pred_odd = pred_odd & (j != NV - 1)
    else:
        pred_even = pred_even & (j != NV - 1)

    @pl.when(pred_even)
    def _p0_even():
        _produce(bufa_ref, tma_ref, masked=False)
        _stats(bufb_ref, tmb_ref)

    @pl.when(pred_odd)
    def _p0_odd():
        _produce(bufb_ref, tmb_ref, masked=False)
        _stats(bufa_ref, tma_ref)

    @pl.when((p == 0) & (j == NV - 1))
    def _p0_edge():
        if EDGE_ODD:
            _produce(bufb_ref, tmb_ref, masked=True)
            _stats(bufa_ref, tma_ref)
        else:
            _produce(bufa_ref, tma_ref, masked=True)
            _stats(bufb_ref, tmb_ref)

    @pl.when((p == 1) & (j == 0))
    def _finalize():
        m_ref[...] = m_ref[...] + jnp.log(s_ref[...])

    @pl.when((p == 1) & (j < NV))
    def _emit():
        out_ref[...] = _logits() - m_ref[...]


def kernel(context_words, emb_table, W1, b1, W2, b2):
    idx = context_words.reshape(B * CTX)
    idx32 = (idx[:, None] * EMB
             + jnp.arange(EMB, dtype=jnp.int32)).reshape(NELEM)
    table_flat = emb_table.reshape(VOCAB * EMB)
    flat = _make_gather()(table_flat, idx32)
    emb = flat.reshape(B, DIN)

    out_t = pl.pallas_call(
        _mlp_lsm_body,
        grid=(2, NV + 1),
        in_specs=[
            pl.BlockSpec((B, DIN), lambda p, j: (0, 0)),
            pl.BlockSpec((HID, DIN), lambda p, j: (0, 0)),
            pl.BlockSpec((1, HID), lambda p, j: (0, 0)),
            pl.BlockSpec((VT, HID), lambda p, j: (jnp.minimum(j, NV - 1), 0)),
            pl.BlockSpec((1, VT), lambda p, j: (0, jnp.minimum(j, NV - 1))),
        ],
        out_specs=pl.BlockSpec(
            (VT, B),
            lambda p, j: (jnp.where(p == 1, jnp.minimum(j, NV - 1), 0), 0)),
        out_shape=jax.ShapeDtypeStruct((VOCAB, B), jnp.float32),
        scratch_shapes=[
            pltpu.VMEM((B, HID), jnp.bfloat16),
            pltpu.VMEM((VT, B), jnp.float32),
            pltpu.VMEM((VT, B), jnp.float32),
            pltpu.VMEM((1, B), jnp.float32),
            pltpu.VMEM((1, B), jnp.float32),
            pltpu.VMEM((1, B), jnp.float32),
            pltpu.VMEM((1, B), jnp.float32),
        ],
        compiler_params=pltpu.CompilerParams(
            dimension_semantics=("arbitrary", "arbitrary")),
    )(emb, W1, b1.reshape(1, HID), W2, b2.reshape(1, VOCAB))
    return out_t.T
